# Initial kernel scaffold; baseline (speedup 1.0000x reference)
import jax
import jax.numpy as jnp
from jax.experimental import pallas as pl


def kernel(feat, segment_ids, W_ih, W_hh, b_ih, b_hh):
    raise NotImplementedError("write your pallas kernel here")


# speedup vs baseline: 5.6401x; 5.6401x over previous
import functools

import jax
import jax.numpy as jnp
from jax.experimental import pallas as pl
from jax.experimental.pallas import tpu as pltpu

_NUM_SEGMENTS = 64
_N_ITERS = 6
_BLK = 512
_NEG = -1e30


def _body(feat_ref, seg_ref, w_ih_ref, w_hh_ref, bias_ref, out_ref,
          h_ref, c_ref, m_ref, l_ref, acc_ref, *, nb, d, n):
    t = pl.program_id(0)
    j = pl.program_id(1)
    f32 = jnp.float32

    @pl.when(j == 0)
    def _start_iter():
        first = t == 0
        h_prev = jnp.where(first, 0.0, h_ref[...])
        c_prev = jnp.where(first, 0.0, c_ref[...])
        l_col = l_ref[...]
        acc = acc_ref[...]
        readout = jnp.where(jnp.logical_and(jnp.logical_not(first), l_col > 0.0),
                            acc / l_col, 0.0)
        q_star = jnp.concatenate([h_prev, readout], axis=1)
        gates = (jax.lax.dot_general(q_star, w_ih_ref[...],
                                     (((1,), (1,)), ((), ())),
                                     preferred_element_type=f32)
                 + jax.lax.dot_general(h_prev, w_hh_ref[...],
                                       (((1,), (1,)), ((), ())),
                                       preferred_element_type=f32)
                 + bias_ref[...])
        i_g = jax.nn.sigmoid(gates[:, 0 * d:1 * d])
        f_g = jax.nn.sigmoid(gates[:, 1 * d:2 * d])
        g_g = jnp.tanh(gates[:, 2 * d:3 * d])
        o_g = jax.nn.sigmoid(gates[:, 3 * d:4 * d])
        c_new = f_g * c_prev + i_g * g_g
        h_new = o_g * jnp.tanh(c_new)
        h_ref[...] = h_new
        c_ref[...] = c_new
        m_ref[...] = jnp.full((_NUM_SEGMENTS, 1), _NEG, f32)
        l_ref[...] = jnp.zeros((_NUM_SEGMENTS, 1), f32)
        acc_ref[...] = jnp.zeros((_NUM_SEGMENTS, d), f32)

    q = h_ref[...]
    seg_row = seg_ref[0]
    row_ids = jax.lax.broadcasted_iota(jnp.int32, (_BLK, 1), 0)
    valid_col = (j * _BLK + row_ids) < n
    fb = jnp.where(valid_col, feat_ref[...], 0.0)

    ids = jax.lax.broadcasted_iota(jnp.int32, (_NUM_SEGMENTS, _BLK), 0)
    s_bool = jnp.broadcast_to(seg_row, (_NUM_SEGMENTS, _BLK)) == ids
    s_f = s_bool.astype(f32)

    hi = jax.lax.Precision.HIGHEST
    e_full = jax.lax.dot_general(q, fb, (((1,), (1,)), ((), ())),
                                 precision=hi, preferred_element_type=f32)
    e_row = jnp.sum(jnp.where(s_bool, e_full, 0.0), axis=0, keepdims=True)

    m_old = m_ref[...]
    e_b = jnp.broadcast_to(e_row, (_NUM_SEGMENTS, _BLK))
    m_blk = jnp.max(jnp.where(s_bool, e_b, _NEG), axis=1, keepdims=True)
    m_new = jnp.maximum(m_old, m_blk)
    scale = jnp.exp(m_old - m_new)
    m_g = jnp.sum(s_f * m_new, axis=0, keepdims=True)
    p_row = jnp.exp(e_row - m_g)

    w = s_f * p_row
    l_new = l_ref[...] * scale + jnp.sum(w, axis=1, keepdims=True)
    acc_blk = jax.lax.dot_general(w, fb, (((1,), (0,)), ((), ())),
                                  precision=hi, preferred_element_type=f32)
    acc_new = acc_ref[...] * scale + acc_blk

    m_ref[...] = m_new
    l_ref[...] = l_new
    acc_ref[...] = acc_new

    @pl.when(jnp.logical_and(t == _N_ITERS - 1, j == nb - 1))
    def _finish():
        readout = jnp.where(l_new > 0.0, acc_new / l_new, 0.0)
        out_ref[...] = jnp.concatenate([h_ref[...], readout], axis=1)


@jax.jit
def kernel(feat, segment_ids, W_ih, W_hh, b_ih, b_hh):
    n, d = feat.shape
    nb = (n + _BLK - 1) // _BLK
    n_pad = nb * _BLK
    seg = segment_ids.astype(jnp.int32)
    seg = jnp.concatenate(
        [seg, jnp.full((n_pad - n,), _NUM_SEGMENTS, jnp.int32)])
    seg = seg.reshape(nb, 1, _BLK)
    bias = (b_ih + b_hh).reshape(1, 4 * d).astype(jnp.float32)

    grid = (_N_ITERS, nb)
    out = pl.pallas_call(
        functools.partial(_body, nb=nb, d=d, n=n),
        grid=grid,
        in_specs=[
            pl.BlockSpec((_BLK, d), lambda t, j: (j, 0)),
            pl.BlockSpec((1, 1, _BLK), lambda t, j: (j, 0, 0)),
            pl.BlockSpec((4 * d, 2 * d), lambda t, j: (0, 0)),
            pl.BlockSpec((4 * d, d), lambda t, j: (0, 0)),
            pl.BlockSpec((1, 4 * d), lambda t, j: (0, 0)),
        ],
        out_specs=pl.BlockSpec((_NUM_SEGMENTS, 2 * d), lambda t, j: (0, 0)),
        out_shape=jax.ShapeDtypeStruct((_NUM_SEGMENTS, 2 * d), jnp.float32),
        scratch_shapes=[
            pltpu.VMEM((_NUM_SEGMENTS, d), jnp.float32),
            pltpu.VMEM((_NUM_SEGMENTS, d), jnp.float32),
            pltpu.VMEM((_NUM_SEGMENTS, 1), jnp.float32),
            pltpu.VMEM((_NUM_SEGMENTS, 1), jnp.float32),
            pltpu.VMEM((_NUM_SEGMENTS, d), jnp.float32),
        ],
        compiler_params=pltpu.CompilerParams(
            dimension_semantics=("arbitrary", "arbitrary")),
    )(feat, seg, W_ih, W_hh, bias)
    return out


# speedup vs baseline: 6.5215x; 1.1563x over previous
import functools

import jax
import jax.numpy as jnp
from jax.experimental import pallas as pl
from jax.experimental.pallas import tpu as pltpu

_NUM_SEGMENTS = 64
_N_ITERS = 6
_BLK = 1024
_NEG = -1e30


def _body(feat_ref, seg_ref, w_ih_ref, w_hh_ref, bias_ref, out_ref,
          h_ref, c_ref, m_ref, l_ref, acc_ref, *, nb, d, n):
    t = pl.program_id(0)
    j = pl.program_id(1)
    f32 = jnp.float32

    @pl.when(j == 0)
    def _start_iter():
        first = t == 0
        h_prev = jnp.where(first, 0.0, h_ref[...])
        c_prev = jnp.where(first, 0.0, c_ref[...])
        l_col = l_ref[...]
        acc = acc_ref[...]
        readout = jnp.where(jnp.logical_and(jnp.logical_not(first), l_col > 0.0),
                            acc / l_col, 0.0)
        q_star = jnp.concatenate([h_prev, readout], axis=1)
        gates = (jax.lax.dot_general(q_star, w_ih_ref[...],
                                     (((1,), (1,)), ((), ())),
                                     preferred_element_type=f32)
                 + jax.lax.dot_general(h_prev, w_hh_ref[...],
                                       (((1,), (1,)), ((), ())),
                                       preferred_element_type=f32)
                 + bias_ref[...])
        i_g = jax.nn.sigmoid(gates[:, 0 * d:1 * d])
        f_g = jax.nn.sigmoid(gates[:, 1 * d:2 * d])
        g_g = jnp.tanh(gates[:, 2 * d:3 * d])
        o_g = jax.nn.sigmoid(gates[:, 3 * d:4 * d])
        c_new = f_g * c_prev + i_g * g_g
        h_new = o_g * jnp.tanh(c_new)
        h_ref[...] = h_new
        c_ref[...] = c_new
        m_ref[...] = jnp.full((_NUM_SEGMENTS, 1), _NEG, f32)
        l_ref[...] = jnp.zeros((_NUM_SEGMENTS, 1), f32)
        acc_ref[...] = jnp.zeros((_NUM_SEGMENTS, d), f32)

    q = h_ref[...]
    seg_row = seg_ref[0]
    row_ids = jax.lax.broadcasted_iota(jnp.int32, (_BLK, 1), 0)
    valid_col = (j * _BLK + row_ids) < n
    fb = jnp.where(valid_col, feat_ref[...], 0.0)

    ids = jax.lax.broadcasted_iota(jnp.int32, (_NUM_SEGMENTS, _BLK), 0)
    s_bool = jnp.broadcast_to(seg_row, (_NUM_SEGMENTS, _BLK)) == ids
    s_f = s_bool.astype(f32)

    hi = jax.lax.Precision.HIGHEST
    e_full = jax.lax.dot_general(q, fb, (((1,), (1,)), ((), ())),
                                 precision=hi, preferred_element_type=f32)
    e_row = jnp.sum(jnp.where(s_bool, e_full, 0.0), axis=0, keepdims=True)

    m_old = m_ref[...]
    m_blk = jnp.max(jnp.where(s_bool, e_full, _NEG), axis=1, keepdims=True)
    m_new = jnp.maximum(m_old, m_blk)
    scale = jnp.exp(m_old - m_new)
    m_g = jnp.sum(s_f * m_new, axis=0, keepdims=True)
    p_row = jnp.exp(e_row - m_g)

    w = s_f * p_row
    l_new = l_ref[...] * scale + jnp.sum(w, axis=1, keepdims=True)
    acc_blk = jax.lax.dot_general(w, fb, (((1,), (0,)), ((), ())),
                                  precision=hi, preferred_element_type=f32)
    acc_new = acc_ref[...] * scale + acc_blk

    m_ref[...] = m_new
    l_ref[...] = l_new
    acc_ref[...] = acc_new

    @pl.when(jnp.logical_and(t == _N_ITERS - 1, j == nb - 1))
    def _finish():
        readout = jnp.where(l_new > 0.0, acc_new / l_new, 0.0)
        out_ref[...] = jnp.concatenate([h_ref[...], readout], axis=1)


@jax.jit
def kernel(feat, segment_ids, W_ih, W_hh, b_ih, b_hh):
    n, d = feat.shape
    nb = (n + _BLK - 1) // _BLK
    n_pad = nb * _BLK
    seg = segment_ids.astype(jnp.int32)
    seg = jnp.concatenate(
        [seg, jnp.full((n_pad - n,), _NUM_SEGMENTS, jnp.int32)])
    seg = seg.reshape(nb, 1, _BLK)
    bias = (b_ih + b_hh).reshape(1, 4 * d).astype(jnp.float32)

    grid = (_N_ITERS, nb)
    out = pl.pallas_call(
        functools.partial(_body, nb=nb, d=d, n=n),
        grid=grid,
        in_specs=[
            pl.BlockSpec((_BLK, d), lambda t, j: (j, 0)),
            pl.BlockSpec((1, 1, _BLK), lambda t, j: (j, 0, 0)),
            pl.BlockSpec((4 * d, 2 * d), lambda t, j: (0, 0)),
            pl.BlockSpec((4 * d, d), lambda t, j: (0, 0)),
            pl.BlockSpec((1, 4 * d), lambda t, j: (0, 0)),
        ],
        out_specs=pl.BlockSpec((_NUM_SEGMENTS, 2 * d), lambda t, j: (0, 0)),
        out_shape=jax.ShapeDtypeStruct((_NUM_SEGMENTS, 2 * d), jnp.float32),
        scratch_shapes=[
            pltpu.VMEM((_NUM_SEGMENTS, d), jnp.float32),
            pltpu.VMEM((_NUM_SEGMENTS, d), jnp.float32),
            pltpu.VMEM((_NUM_SEGMENTS, 1), jnp.float32),
            pltpu.VMEM((_NUM_SEGMENTS, 1), jnp.float32),
            pltpu.VMEM((_NUM_SEGMENTS, d), jnp.float32),
        ],
        compiler_params=pltpu.CompilerParams(
            dimension_semantics=("arbitrary", "arbitrary")),
    )(feat, seg, W_ih, W_hh, bias)
    return out


# speedup vs baseline: 11.8996x; 1.8247x over previous
import functools

import jax
import jax.numpy as jnp
from jax.experimental import pallas as pl
from jax.experimental.pallas import tpu as pltpu

_NUM_SEGMENTS = 64
_N_ITERS = 6
_BLK = 1024
_NEG = -1e30


def _body(feat_ref, seg_ref, w_ih_ref, w_hh_ref, bias_ref, out_ref,
          h_ref, c_ref, m_ref, l_ref, acc_ref, *, nb, d, n):
    t = pl.program_id(0)
    j = pl.program_id(1)
    f32 = jnp.float32

    @pl.when(j == 0)
    def _start_iter():
        first = t == 0
        h_prev = jnp.where(first, 0.0, h_ref[...])
        c_prev = jnp.where(first, 0.0, c_ref[...])
        l_col = l_ref[...]
        acc = acc_ref[...]
        readout = jnp.where(jnp.logical_and(jnp.logical_not(first), l_col > 0.0),
                            acc / l_col, 0.0)
        q_star = jnp.concatenate([h_prev, readout], axis=1)
        gates = (jax.lax.dot_general(q_star, w_ih_ref[...],
                                     (((1,), (1,)), ((), ())),
                                     preferred_element_type=f32)
                 + jax.lax.dot_general(h_prev, w_hh_ref[...],
                                       (((1,), (1,)), ((), ())),
                                       preferred_element_type=f32)
                 + bias_ref[...])
        i_g = jax.nn.sigmoid(gates[:, 0 * d:1 * d])
        f_g = jax.nn.sigmoid(gates[:, 1 * d:2 * d])
        g_g = jnp.tanh(gates[:, 2 * d:3 * d])
        o_g = jax.nn.sigmoid(gates[:, 3 * d:4 * d])
        c_new = f_g * c_prev + i_g * g_g
        h_new = o_g * jnp.tanh(c_new)
        h_ref[...] = h_new
        c_ref[...] = c_new
        m_ref[...] = jnp.full((_NUM_SEGMENTS, 1), _NEG, f32)
        l_ref[...] = jnp.zeros((_NUM_SEGMENTS, 1), f32)
        acc_ref[...] = jnp.zeros((_NUM_SEGMENTS, d), f32)

    q = h_ref[...]
    seg_row = seg_ref[0]
    row_ids = jax.lax.broadcasted_iota(jnp.int32, (_BLK, 1), 0)
    valid_col = (j * _BLK + row_ids) < n
    fb = jnp.where(valid_col, feat_ref[...], 0.0)

    ids = jax.lax.broadcasted_iota(jnp.int32, (_NUM_SEGMENTS, _BLK), 0)
    s_bool = jnp.broadcast_to(seg_row, (_NUM_SEGMENTS, _BLK)) == ids
    s_f = s_bool.astype(f32)

    bf16 = jnp.bfloat16
    dims_e = (((1,), (1,)), ((), ()))
    fb_hi = fb.astype(bf16)
    fb_lo = (fb - fb_hi.astype(f32)).astype(bf16)
    q_hi = q.astype(bf16)
    q_lo = (q - q_hi.astype(f32)).astype(bf16)
    e_full = (jax.lax.dot_general(q_hi, fb_hi, dims_e,
                                  preferred_element_type=f32)
              + jax.lax.dot_general(q_hi, fb_lo, dims_e,
                                    preferred_element_type=f32)
              + jax.lax.dot_general(q_lo, fb_hi, dims_e,
                                    preferred_element_type=f32))
    e_row = jnp.sum(jnp.where(s_bool, e_full, 0.0), axis=0, keepdims=True)

    m_old = m_ref[...]
    m_blk = jnp.max(jnp.where(s_bool, e_full, _NEG), axis=1, keepdims=True)
    m_new = jnp.maximum(m_old, m_blk)
    scale = jnp.exp(m_old - m_new)
    m_g = jnp.sum(s_f * m_new, axis=0, keepdims=True)
    p_row = jnp.exp(e_row - m_g)

    w = s_f * p_row
    l_new = l_ref[...] * scale + jnp.sum(w, axis=1, keepdims=True)
    dims_a = (((1,), (0,)), ((), ()))
    w_hi = w.astype(bf16)
    acc_blk = (jax.lax.dot_general(w_hi, fb_hi, dims_a,
                                   preferred_element_type=f32)
               + jax.lax.dot_general(w_hi, fb_lo, dims_a,
                                     preferred_element_type=f32))
    acc_new = acc_ref[...] * scale + acc_blk

    m_ref[...] = m_new
    l_ref[...] = l_new
    acc_ref[...] = acc_new

    @pl.when(jnp.logical_and(t == _N_ITERS - 1, j == nb - 1))
    def _finish():
        readout = jnp.where(l_new > 0.0, acc_new / l_new, 0.0)
        out_ref[...] = jnp.concatenate([h_ref[...], readout], axis=1)


@jax.jit
def kernel(feat, segment_ids, W_ih, W_hh, b_ih, b_hh):
    n, d = feat.shape
    nb = (n + _BLK - 1) // _BLK
    n_pad = nb * _BLK
    seg = segment_ids.astype(jnp.int32)
    seg = jnp.concatenate(
        [seg, jnp.full((n_pad - n,), _NUM_SEGMENTS, jnp.int32)])
    seg = seg.reshape(nb, 1, _BLK)
    bias = (b_ih + b_hh).reshape(1, 4 * d).astype(jnp.float32)

    grid = (_N_ITERS, nb)
    out = pl.pallas_call(
        functools.partial(_body, nb=nb, d=d, n=n),
        grid=grid,
        in_specs=[
            pl.BlockSpec((_BLK, d), lambda t, j: (j, 0)),
            pl.BlockSpec((1, 1, _BLK), lambda t, j: (j, 0, 0)),
            pl.BlockSpec((4 * d, 2 * d), lambda t, j: (0, 0)),
            pl.BlockSpec((4 * d, d), lambda t, j: (0, 0)),
            pl.BlockSpec((1, 4 * d), lambda t, j: (0, 0)),
        ],
        out_specs=pl.BlockSpec((_NUM_SEGMENTS, 2 * d), lambda t, j: (0, 0)),
        out_shape=jax.ShapeDtypeStruct((_NUM_SEGMENTS, 2 * d), jnp.float32),
        scratch_shapes=[
            pltpu.VMEM((_NUM_SEGMENTS, d), jnp.float32),
            pltpu.VMEM((_NUM_SEGMENTS, d), jnp.float32),
            pltpu.VMEM((_NUM_SEGMENTS, 1), jnp.float32),
            pltpu.VMEM((_NUM_SEGMENTS, 1), jnp.float32),
            pltpu.VMEM((_NUM_SEGMENTS, d), jnp.float32),
        ],
        compiler_params=pltpu.CompilerParams(
            dimension_semantics=("arbitrary", "arbitrary")),
    )(feat, seg, W_ih, W_hh, bias)
    return out


# speedup vs baseline: 14.2870x; 1.2006x over previous
import functools

import jax
import jax.numpy as jnp
from jax.experimental import pallas as pl
from jax.experimental.pallas import tpu as pltpu

_NUM_SEGMENTS = 64
_N_ITERS = 6
_BLK = 1024
_NEG = -1e30


def _body(feat_ref, seg_ref, w_ih_ref, w_hh_ref, bias_ref, out_ref,
          h_ref, c_ref, m_ref, l_ref, acc_ref, *, nb, d, n):
    t = pl.program_id(0)
    j = pl.program_id(1)
    f32 = jnp.float32

    @pl.when(j == 0)
    def _start_iter():
        first = t == 0
        h_prev = jnp.where(first, 0.0, h_ref[...])
        c_prev = jnp.where(first, 0.0, c_ref[...])
        l_col = l_ref[...]
        acc = acc_ref[...]
        readout = jnp.where(jnp.logical_and(jnp.logical_not(first), l_col > 0.0),
                            acc / l_col, 0.0)
        q_star = jnp.concatenate([h_prev, readout], axis=1)
        gates = (jax.lax.dot_general(q_star, w_ih_ref[...],
                                     (((1,), (1,)), ((), ())),
                                     preferred_element_type=f32)
                 + jax.lax.dot_general(h_prev, w_hh_ref[...],
                                       (((1,), (1,)), ((), ())),
                                       preferred_element_type=f32)
                 + bias_ref[...])
        i_g = jax.nn.sigmoid(gates[:, 0 * d:1 * d])
        f_g = jax.nn.sigmoid(gates[:, 1 * d:2 * d])
        g_g = jnp.tanh(gates[:, 2 * d:3 * d])
        o_g = jax.nn.sigmoid(gates[:, 3 * d:4 * d])
        c_new = f_g * c_prev + i_g * g_g
        h_new = o_g * jnp.tanh(c_new)
        h_ref[...] = h_new
        c_ref[...] = c_new
        m_ref[...] = jnp.full((_NUM_SEGMENTS, 1), _NEG, f32)
        l_ref[...] = jnp.zeros((_NUM_SEGMENTS, 1), f32)
        acc_ref[...] = jnp.zeros((_NUM_SEGMENTS, d), f32)

    q = h_ref[...]
    seg_row = seg_ref[0]
    row_ids = jax.lax.broadcasted_iota(jnp.int32, (_BLK, 1), 0)
    valid_col = (j * _BLK + row_ids) < n
    fb = jnp.where(valid_col, feat_ref[...], 0.0)

    ids = jax.lax.broadcasted_iota(jnp.int32, (_NUM_SEGMENTS, _BLK), 0)
    s_bool = jnp.broadcast_to(seg_row, (_NUM_SEGMENTS, _BLK)) == ids

    bf16 = jnp.bfloat16
    dims_e = (((1,), (1,)), ((), ()))
    fb_hi = fb.astype(bf16)
    fb_lo = (fb - fb_hi.astype(f32)).astype(bf16)
    q_hi = q.astype(bf16)
    q_lo = (q - q_hi.astype(f32)).astype(bf16)
    d1 = jax.lax.dot_general(q_hi, fb_hi, dims_e, preferred_element_type=f32)
    d2 = jax.lax.dot_general(q_hi, fb_lo, dims_e, preferred_element_type=f32)
    d3 = jax.lax.dot_general(q_lo, fb_hi, dims_e, preferred_element_type=f32)

    m_old = m_ref[...]
    m_blk = jnp.max(jnp.where(s_bool, d1, _NEG), axis=1, keepdims=True)
    m_new = jnp.maximum(m_old, m_blk)
    scale = jnp.exp(m_old - m_new)

    e_full = d1 + d2 + d3
    z = jnp.where(s_bool, e_full - m_new, 0.0)
    p_row = jnp.exp(jnp.sum(z, axis=0, keepdims=True))

    w = jnp.where(s_bool, p_row, 0.0)
    l_new = l_ref[...] * scale + jnp.sum(w, axis=1, keepdims=True)
    dims_a = (((1,), (0,)), ((), ()))
    w_hi = w.astype(bf16)
    acc_blk = (jax.lax.dot_general(w_hi, fb_hi, dims_a,
                                   preferred_element_type=f32)
               + jax.lax.dot_general(w_hi, fb_lo, dims_a,
                                     preferred_element_type=f32))
    acc_new = acc_ref[...] * scale + acc_blk

    m_ref[...] = m_new
    l_ref[...] = l_new
    acc_ref[...] = acc_new

    @pl.when(jnp.logical_and(t == _N_ITERS - 1, j == nb - 1))
    def _finish():
        readout = jnp.where(l_new > 0.0, acc_new / l_new, 0.0)
        out_ref[...] = jnp.concatenate([h_ref[...], readout], axis=1)


@jax.jit
def kernel(feat, segment_ids, W_ih, W_hh, b_ih, b_hh):
    n, d = feat.shape
    nb = (n + _BLK - 1) // _BLK
    n_pad = nb * _BLK
    seg = segment_ids.astype(jnp.int32)
    seg = jnp.concatenate(
        [seg, jnp.full((n_pad - n,), _NUM_SEGMENTS, jnp.int32)])
    seg = seg.reshape(nb, 1, _BLK)
    bias = (b_ih + b_hh).reshape(1, 4 * d).astype(jnp.float32)

    grid = (_N_ITERS, nb)
    out = pl.pallas_call(
        functools.partial(_body, nb=nb, d=d, n=n),
        grid=grid,
        in_specs=[
            pl.BlockSpec((_BLK, d), lambda t, j: (j, 0)),
            pl.BlockSpec((1, 1, _BLK), lambda t, j: (j, 0, 0)),
            pl.BlockSpec((4 * d, 2 * d), lambda t, j: (0, 0)),
            pl.BlockSpec((4 * d, d), lambda t, j: (0, 0)),
            pl.BlockSpec((1, 4 * d), lambda t, j: (0, 0)),
        ],
        out_specs=pl.BlockSpec((_NUM_SEGMENTS, 2 * d), lambda t, j: (0, 0)),
        out_shape=jax.ShapeDtypeStruct((_NUM_SEGMENTS, 2 * d), jnp.float32),
        scratch_shapes=[
            pltpu.VMEM((_NUM_SEGMENTS, d), jnp.float32),
            pltpu.VMEM((_NUM_SEGMENTS, d), jnp.float32),
            pltpu.VMEM((_NUM_SEGMENTS, 1), jnp.float32),
            pltpu.VMEM((_NUM_SEGMENTS, 1), jnp.float32),
            pltpu.VMEM((_NUM_SEGMENTS, d), jnp.float32),
        ],
        compiler_params=pltpu.CompilerParams(
            dimension_semantics=("arbitrary", "arbitrary")),
    )(feat, seg, W_ih, W_hh, bias)
    return out
